# Initial kernel scaffold; baseline (speedup 1.0000x reference)
import jax
import jax.numpy as jnp
from jax.experimental import pallas as pl


def kernel(x, edge_index, W1, b1, W2, b2):
    raise NotImplementedError("write your pallas kernel here")


# speedup vs baseline: 25.3817x; 25.3817x over previous
import functools

import jax
import jax.numpy as jnp
from jax import lax
from jax.experimental import pallas as pl
from jax.experimental.pallas import tpu as pltpu
from jax.experimental.pallas import tpu_sc as plsc

N_NODES = 10000
NPAD = 10240
PAD_ROW = 10000
N_EDGES = 320000
WIN = 128
NC, NS = 2, 16
EPAD = ((N_EDGES + NC * NS * WIN - 1) // (NC * NS * WIN)) * (NC * NS * WIN)
RPS = NPAD // NS
BLK = 1024
GRID = NPAD // BLK

_mesh = plsc.VectorSubcoreMesh(core_axis_name="core", subcore_axis_name="subcore")
_sc_params = pltpu.CompilerParams(use_tc_tiling_on_sc=False)


def _sc_degree(dst_w):
    @functools.partial(
        pl.kernel,
        out_type=jax.ShapeDtypeStruct((NC, NPAD), jnp.float32),
        mesh=_mesh,
        compiler_params=_sc_params,
        scratch_types=[
            pltpu.VMEM_SHARED((NPAD,), jnp.float32),
            pltpu.VMEM((RPS,), jnp.float32),
            pltpu.VMEM((WIN,), jnp.float32),
        ],
    )
    def k(dst_hbm, out_hbm, acc, zbuf, ones_v):
        cid = lax.axis_index("core")
        sid = lax.axis_index("subcore")

        @pl.loop(0, RPS // 16)
        def _(i):
            zbuf[pl.ds(i * 16, 16)] = jnp.zeros((16,), jnp.float32)

        pltpu.sync_copy(zbuf, acc.at[pl.ds(sid * RPS, RPS)])

        @pl.loop(0, WIN // 16)
        def _(i):
            ones_v[pl.ds(i * 16, 16)] = jnp.ones((16,), jnp.float32)

        plsc.subcore_barrier()

        def body(d_v):
            pltpu.sync_copy(ones_v, acc.at[d_v.at[0]], add=True)

        pltpu.emit_pipeline(
            body,
            grid=(EPAD // WIN,),
            in_specs=[pl.BlockSpec((1, WIN), index_map=lambda i: (0, i))],
            out_specs=[],
            core_axis_name=("core", "subcore"),
            dimension_semantics=(pltpu.PARALLEL,),
        )(dst_hbm)
        plsc.subcore_barrier()
        pltpu.sync_copy(
            acc.at[pl.ds(sid * RPS, RPS)], out_hbm.at[cid, pl.ds(sid * RPS, RPS)]
        )

    return k(dst_w)


def _sc_message(y, src_w, dst_w, zeros_hbm, d):
    @functools.partial(
        pl.kernel,
        out_type=jax.ShapeDtypeStruct((NC, NPAD, d), jnp.float32),
        mesh=_mesh,
        compiler_params=_sc_params,
        scratch_types=[
            pltpu.VMEM_SHARED((NPAD, d), jnp.float32),
            pltpu.VMEM((WIN, d), jnp.float32),
        ],
    )
    def k(y_hbm, src_hbm, dst_hbm, z_hbm, out_hbm, acc, rows_v):
        cid = lax.axis_index("core")
        sid = lax.axis_index("subcore")
        pltpu.sync_copy(
            z_hbm.at[pl.ds(sid * RPS, RPS)], acc.at[pl.ds(sid * RPS, RPS)]
        )
        plsc.subcore_barrier()

        def body(s_v, d_v):
            pltpu.sync_copy(y_hbm.at[s_v.at[0]], rows_v)
            pltpu.sync_copy(rows_v, acc.at[d_v.at[0]], add=True)

        pltpu.emit_pipeline(
            body,
            grid=(EPAD // WIN,),
            in_specs=[
                pl.BlockSpec((1, WIN), index_map=lambda i: (0, i)),
                pl.BlockSpec((1, WIN), index_map=lambda i: (0, i)),
            ],
            out_specs=[],
            core_axis_name=("core", "subcore"),
            dimension_semantics=(pltpu.PARALLEL,),
        )(src_hbm, dst_hbm)
        plsc.subcore_barrier()
        pltpu.sync_copy(
            acc.at[pl.ds(sid * RPS, RPS)], out_hbm.at[cid, pl.ds(sid * RPS, RPS)]
        )

    return k(y, src_w, dst_w, zeros_hbm)


def _xw_kernel(x_ref, w_ref, o_ref):
    o_ref[...] = jnp.dot(
        x_ref[...], w_ref[...], preferred_element_type=jnp.float32,
        precision=lax.Precision.HIGHEST,
    )


def _tc_xw(x, W1):
    return pl.pallas_call(
        _xw_kernel,
        grid=(GRID,),
        in_specs=[
            pl.BlockSpec((BLK, 128), lambda i: (i, 0)),
            pl.BlockSpec((128, 16), lambda i: (0, 0)),
        ],
        out_specs=pl.BlockSpec((BLK, 16), lambda i: (i, 0)),
        out_shape=jax.ShapeDtypeStruct((NPAD, 16), jnp.float32),
    )(x, W1)


def _scale_kernel(xw_ref, d0_ref, d1_ref, y_ref, dinv_ref):
    deg = d0_ref[...] + d1_ref[...] + 1.0
    dinv = lax.rsqrt(deg)
    dinv_ref[...] = dinv
    y_ref[...] = xw_ref[...] * dinv


def _tc_scale(xw, d0, d1):
    return pl.pallas_call(
        _scale_kernel,
        grid=(GRID,),
        in_specs=[
            pl.BlockSpec((BLK, 16), lambda i: (i, 0)),
            pl.BlockSpec((BLK, 1), lambda i: (i, 0)),
            pl.BlockSpec((BLK, 1), lambda i: (i, 0)),
        ],
        out_specs=[
            pl.BlockSpec((BLK, 16), lambda i: (i, 0)),
            pl.BlockSpec((BLK, 1), lambda i: (i, 0)),
        ],
        out_shape=[
            jax.ShapeDtypeStruct((NPAD, 16), jnp.float32),
            jax.ShapeDtypeStruct((NPAD, 1), jnp.float32),
        ],
    )(xw, d0, d1)


def _mid_kernel(s0_ref, s1_ref, y1_ref, dinv_ref, b1_ref, w2_ref, y2_ref):
    dinv = dinv_ref[...]
    h = dinv * (s0_ref[0] + s1_ref[0] + y1_ref[...]) + b1_ref[...]
    h = jnp.maximum(h, 0.0)
    y2_ref[...] = jnp.dot(
        h, w2_ref[...], preferred_element_type=jnp.float32,
        precision=lax.Precision.HIGHEST,
    ) * dinv


def _tc_mid(s1p, y1, dinv, b1, W2):
    return pl.pallas_call(
        _mid_kernel,
        grid=(GRID,),
        in_specs=[
            pl.BlockSpec((1, BLK, 16), lambda i: (0, i, 0)),
            pl.BlockSpec((1, BLK, 16), lambda i: (1, i, 0)),
            pl.BlockSpec((BLK, 16), lambda i: (i, 0)),
            pl.BlockSpec((BLK, 1), lambda i: (i, 0)),
            pl.BlockSpec((1, 16), lambda i: (0, 0)),
            pl.BlockSpec((16, 40), lambda i: (0, 0)),
        ],
        out_specs=pl.BlockSpec((BLK, 40), lambda i: (i, 0)),
        out_shape=jax.ShapeDtypeStruct((NPAD, 40), jnp.float32),
    )(s1p, s1p, y1, dinv, b1, W2)


def _post_kernel(s0_ref, s1_ref, y2_ref, dinv_ref, b2_ref, o_ref):
    z = dinv_ref[...] * (s0_ref[0] + s1_ref[0] + y2_ref[...]) + b2_ref[...]
    m = jnp.max(z, axis=1, keepdims=True)
    zs = z - m
    lse = jnp.log(jnp.sum(jnp.exp(zs), axis=1, keepdims=True))
    o_ref[...] = zs - lse


def _tc_post(s2p, y2, dinv, b2):
    return pl.pallas_call(
        _post_kernel,
        grid=(GRID,),
        in_specs=[
            pl.BlockSpec((1, BLK, 40), lambda i: (0, i, 0)),
            pl.BlockSpec((1, BLK, 40), lambda i: (1, i, 0)),
            pl.BlockSpec((BLK, 40), lambda i: (i, 0)),
            pl.BlockSpec((BLK, 1), lambda i: (i, 0)),
            pl.BlockSpec((1, 40), lambda i: (0, 0)),
        ],
        out_specs=pl.BlockSpec((BLK, 40), lambda i: (i, 0)),
        out_shape=jax.ShapeDtypeStruct((NPAD, 40), jnp.float32),
    )(s2p, s2p, y2, dinv, b2)


def kernel(x, edge_index, W1, b1, W2, b2):
    src = edge_index[0].astype(jnp.int32)
    dst = edge_index[1].astype(jnp.int32)
    src_w = jnp.pad(src, (0, EPAD - N_EDGES)).reshape(1, EPAD)
    dst_w = jnp.pad(dst, (0, EPAD - N_EDGES),
                    constant_values=PAD_ROW).reshape(1, EPAD)
    x_p = jnp.pad(x, ((0, NPAD - N_NODES), (0, 0)))
    z16 = jnp.zeros((NPAD, 16), jnp.float32)
    z40 = jnp.zeros((NPAD, 40), jnp.float32)

    xw = _tc_xw(x_p, W1)
    degp = _sc_degree(dst_w)
    y1, dinv = _tc_scale(xw, degp[0][:, None], degp[1][:, None])
    s1p = _sc_message(y1, src_w, dst_w, z16, 16)
    y2 = _tc_mid(s1p, y1, dinv, b1.reshape(1, 16), W2)
    s2p = _sc_message(y2, src_w, dst_w, z40, 40)
    out = _tc_post(s2p, y2, dinv, b2.reshape(1, 40))
    return out[:N_NODES]


# speedup vs baseline: 27.9844x; 1.1025x over previous
import functools

import jax
import jax.numpy as jnp
from jax import lax
from jax.experimental import pallas as pl
from jax.experimental.pallas import tpu as pltpu
from jax.experimental.pallas import tpu_sc as plsc

N_NODES = 10000
NPAD = 10240
PAD_ROW = 10000
N_EDGES = 320000
WIN = 128
NC, NS = 2, 16
NW = NC * NS
WPS = 80
EPAD = NW * WPS * WIN
RPS = NPAD // NS
NBUF = 4
BLK = 1000
GRID = N_NODES // BLK

_mesh = plsc.VectorSubcoreMesh(core_axis_name="core", subcore_axis_name="subcore")
_sc_params = pltpu.CompilerParams(use_tc_tiling_on_sc=False)


def _sc_degree(dst_w):
    @functools.partial(
        pl.kernel,
        out_type=jax.ShapeDtypeStruct((NC, NPAD), jnp.float32),
        mesh=_mesh,
        compiler_params=_sc_params,
        scratch_types=[
            pltpu.VMEM_SHARED((NPAD,), jnp.float32),
            pltpu.VMEM((WPS, WIN), jnp.int32),
            pltpu.VMEM((RPS,), jnp.float32),
            pltpu.VMEM((WIN,), jnp.float32),
            pltpu.SemaphoreType.DMA,
        ],
    )
    def k(dst_hbm, out_hbm, acc, dsts, zbuf, ones_v, ssem):
        cid = lax.axis_index("core")
        sid = lax.axis_index("subcore")
        w = cid * NS + sid
        pltpu.sync_copy(dst_hbm.at[w], dsts)

        @pl.loop(0, RPS // 16)
        def _(i):
            zbuf[pl.ds(i * 16, 16)] = jnp.zeros((16,), jnp.float32)

        pltpu.sync_copy(zbuf, acc.at[pl.ds(sid * RPS, RPS)])

        @pl.loop(0, WIN // 16)
        def _(i):
            ones_v[pl.ds(i * 16, 16)] = jnp.ones((16,), jnp.float32)

        plsc.subcore_barrier()

        @pl.loop(0, WPS)
        def _(j):
            pltpu.async_copy(ones_v, acc.at[dsts.at[j]], ssem, add=True)

        @pl.loop(0, WPS)
        def _(j):
            pltpu.make_async_copy(ones_v, acc.at[dsts.at[0]], ssem).wait()

        plsc.subcore_barrier()
        pltpu.sync_copy(
            acc.at[pl.ds(sid * RPS, RPS)], out_hbm.at[cid, pl.ds(sid * RPS, RPS)]
        )

    return k(dst_w)


def _sc_message(y, src_w, dst_w, zeros_hbm, d):
    @functools.partial(
        pl.kernel,
        out_type=jax.ShapeDtypeStruct((NC, NPAD, d), jnp.float32),
        mesh=_mesh,
        compiler_params=_sc_params,
        scratch_types=[
            pltpu.VMEM_SHARED((NPAD, d), jnp.float32),
            pltpu.VMEM((WPS, WIN), jnp.int32),
            pltpu.VMEM((WPS, WIN), jnp.int32),
            pltpu.VMEM((NBUF, WIN, d), jnp.float32),
        ] + [pltpu.SemaphoreType.DMA] * (2 * NBUF),
    )
    def k(y_hbm, src_hbm, dst_hbm, z_hbm, out_hbm, acc, srcs, dsts, rows, *sems):
        gsem, ssem = sems[:NBUF], sems[NBUF:]
        cid = lax.axis_index("core")
        sid = lax.axis_index("subcore")
        w = cid * NS + sid
        pltpu.sync_copy(src_hbm.at[w], srcs)
        pltpu.sync_copy(dst_hbm.at[w], dsts)
        pltpu.sync_copy(
            z_hbm.at[pl.ds(sid * RPS, RPS)], acc.at[pl.ds(sid * RPS, RPS)]
        )
        plsc.subcore_barrier()

        def g_start(j, b):
            pltpu.async_copy(y_hbm.at[srcs.at[j]], rows.at[b], gsem[b])

        def g_wait(b):
            pltpu.make_async_copy(y_hbm.at[srcs.at[0]], rows.at[b], gsem[b]).wait()

        def s_start(j, b):
            pltpu.async_copy(rows.at[b], acc.at[dsts.at[j]], ssem[b], add=True)

        def s_wait(b):
            pltpu.make_async_copy(rows.at[b], acc.at[dsts.at[0]], ssem[b]).wait()

        for b in range(NBUF):
            g_start(b, b)

        @pl.loop(0, WPS - NBUF, step=NBUF)
        def _(j):
            for b in range(NBUF):
                g_wait(b)
                s_start(j + b, b)
            for b in range(NBUF):
                s_wait(b)
                g_start(j + NBUF + b, b)

        for b in range(NBUF):
            g_wait(b)
            s_start(WPS - NBUF + b, b)
        for b in range(NBUF):
            s_wait(b)

        plsc.subcore_barrier()
        pltpu.sync_copy(
            acc.at[pl.ds(sid * RPS, RPS)], out_hbm.at[cid, pl.ds(sid * RPS, RPS)]
        )

    return k(y, src_w, dst_w, zeros_hbm)


def _pre_kernel(x_ref, w_ref, d0_ref, d1_ref, y_ref, dinv_ref):
    deg = d0_ref[...] + d1_ref[...] + 1.0
    dinv = lax.rsqrt(deg)
    dinv_ref[...] = dinv
    y_ref[...] = jnp.dot(
        x_ref[...], w_ref[...], preferred_element_type=jnp.float32,
        precision=lax.Precision.HIGHEST,
    ) * dinv


def _tc_pre(x, W1, d0, d1):
    return pl.pallas_call(
        _pre_kernel,
        grid=(GRID,),
        in_specs=[
            pl.BlockSpec((BLK, 128), lambda i: (i, 0)),
            pl.BlockSpec((128, 16), lambda i: (0, 0)),
            pl.BlockSpec((BLK, 1), lambda i: (i, 0)),
            pl.BlockSpec((BLK, 1), lambda i: (i, 0)),
        ],
        out_specs=[
            pl.BlockSpec((BLK, 16), lambda i: (i, 0)),
            pl.BlockSpec((BLK, 1), lambda i: (i, 0)),
        ],
        out_shape=[
            jax.ShapeDtypeStruct((N_NODES, 16), jnp.float32),
            jax.ShapeDtypeStruct((N_NODES, 1), jnp.float32),
        ],
    )(x, W1, d0, d1)


def _mid_kernel(s0_ref, s1_ref, y1_ref, dinv_ref, b1_ref, w2_ref, y2_ref):
    dinv = dinv_ref[...]
    h = dinv * (s0_ref[0] + s1_ref[0] + y1_ref[...]) + b1_ref[...]
    h = jnp.maximum(h, 0.0)
    y2_ref[...] = jnp.dot(
        h, w2_ref[...], preferred_element_type=jnp.float32,
        precision=lax.Precision.HIGHEST,
    ) * dinv


def _tc_mid(s1p, y1, dinv, b1, W2):
    return pl.pallas_call(
        _mid_kernel,
        grid=(GRID,),
        in_specs=[
            pl.BlockSpec((1, BLK, 16), lambda i: (0, i, 0)),
            pl.BlockSpec((1, BLK, 16), lambda i: (1, i, 0)),
            pl.BlockSpec((BLK, 16), lambda i: (i, 0)),
            pl.BlockSpec((BLK, 1), lambda i: (i, 0)),
            pl.BlockSpec((1, 16), lambda i: (0, 0)),
            pl.BlockSpec((16, 40), lambda i: (0, 0)),
        ],
        out_specs=pl.BlockSpec((BLK, 40), lambda i: (i, 0)),
        out_shape=jax.ShapeDtypeStruct((N_NODES, 40), jnp.float32),
    )(s1p, s1p, y1, dinv, b1, W2)


def _post_kernel(s0_ref, s1_ref, y2_ref, dinv_ref, b2_ref, o_ref):
    z = dinv_ref[...] * (s0_ref[0] + s1_ref[0] + y2_ref[...]) + b2_ref[...]
    m = jnp.max(z, axis=1, keepdims=True)
    zs = z - m
    lse = jnp.log(jnp.sum(jnp.exp(zs), axis=1, keepdims=True))
    o_ref[...] = zs - lse


def _tc_post(s2p, y2, dinv, b2):
    return pl.pallas_call(
        _post_kernel,
        grid=(GRID,),
        in_specs=[
            pl.BlockSpec((1, BLK, 40), lambda i: (0, i, 0)),
            pl.BlockSpec((1, BLK, 40), lambda i: (1, i, 0)),
            pl.BlockSpec((BLK, 40), lambda i: (i, 0)),
            pl.BlockSpec((BLK, 1), lambda i: (i, 0)),
            pl.BlockSpec((1, 40), lambda i: (0, 0)),
        ],
        out_specs=pl.BlockSpec((BLK, 40), lambda i: (i, 0)),
        out_shape=jax.ShapeDtypeStruct((N_NODES, 40), jnp.float32),
    )(s2p, s2p, y2, dinv, b2)


def kernel(x, edge_index, W1, b1, W2, b2):
    src = edge_index[0].astype(jnp.int32)
    dst = edge_index[1].astype(jnp.int32)
    src_w = jnp.pad(src, (0, EPAD - N_EDGES)).reshape(NW, WPS, WIN)
    dst_w = jnp.pad(dst, (0, EPAD - N_EDGES),
                    constant_values=PAD_ROW).reshape(NW, WPS, WIN)
    z16 = jnp.zeros((NPAD, 16), jnp.float32)
    z40 = jnp.zeros((NPAD, 40), jnp.float32)

    degp = _sc_degree(dst_w)
    y1, dinv = _tc_pre(x, W1, degp[0, :N_NODES, None], degp[1, :N_NODES, None])
    s1p = _sc_message(y1, src_w, dst_w, z16, 16)
    y2 = _tc_mid(s1p, y1, dinv, b1.reshape(1, 16), W2)
    s2p = _sc_message(y2, src_w, dst_w, z40, 40)
    return _tc_post(s2p, y2, dinv, b2.reshape(1, 40))


# speedup vs baseline: 37.0706x; 1.3247x over previous
import functools

import jax
import jax.numpy as jnp
from jax import lax
from jax.experimental import pallas as pl
from jax.experimental.pallas import tpu as pltpu
from jax.experimental.pallas import tpu_sc as plsc

N_NODES = 10000
NPAD = 10240
PAD_ROW = 10000
N_EDGES = 320000
WIN = 128
NC, NS = 2, 16
NW = NC * NS
WPS = 80
EPAD = NW * WPS * WIN
RPS = NPAD // NS
NBUF = 4
BLK = 1000
GRID = N_NODES // BLK

_mesh = plsc.VectorSubcoreMesh(core_axis_name="core", subcore_axis_name="subcore")
_sc_params = pltpu.CompilerParams(use_tc_tiling_on_sc=False)


def _sc_degree(dst_w):
    @functools.partial(
        pl.kernel,
        out_type=jax.ShapeDtypeStruct((NC, NPAD), jnp.float32),
        mesh=_mesh,
        compiler_params=_sc_params,
        scratch_types=[
            pltpu.VMEM_SHARED((NPAD,), jnp.float32),
            pltpu.VMEM((WPS, WIN), jnp.int32),
            pltpu.VMEM((RPS,), jnp.float32),
            pltpu.VMEM((WIN,), jnp.float32),
            pltpu.SemaphoreType.DMA,
        ],
    )
    def k(dst_hbm, out_hbm, acc, dsts, zbuf, ones_v, ssem):
        cid = lax.axis_index("core")
        sid = lax.axis_index("subcore")
        w = cid * NS + sid
        pltpu.sync_copy(dst_hbm.at[w], dsts)

        @pl.loop(0, RPS // 16)
        def _(i):
            zbuf[pl.ds(i * 16, 16)] = jnp.zeros((16,), jnp.float32)

        pltpu.sync_copy(zbuf, acc.at[pl.ds(sid * RPS, RPS)])

        @pl.loop(0, WIN // 16)
        def _(i):
            ones_v[pl.ds(i * 16, 16)] = jnp.ones((16,), jnp.float32)

        plsc.subcore_barrier()

        @pl.loop(0, WPS)
        def _(j):
            pltpu.async_copy(ones_v, acc.at[dsts.at[j]], ssem, add=True)

        @pl.loop(0, WPS)
        def _(j):
            pltpu.make_async_copy(ones_v, acc.at[dsts.at[0]], ssem).wait()

        plsc.subcore_barrier()
        pltpu.sync_copy(
            acc.at[pl.ds(sid * RPS, RPS)], out_hbm.at[cid, pl.ds(sid * RPS, RPS)]
        )

    return k(dst_w)


def _sc_message(y, src_w, dst_w, zeros_hbm, d):
    @functools.partial(
        pl.kernel,
        out_type=jax.ShapeDtypeStruct((NC, NPAD, d), jnp.float32),
        mesh=_mesh,
        compiler_params=_sc_params,
        scratch_types=[
            pltpu.VMEM_SHARED((NPAD, d), jnp.float32),
            pltpu.VMEM((WPS, WIN), jnp.int32),
            pltpu.VMEM((WPS, WIN), jnp.int32),
            pltpu.VMEM((NBUF, WIN, d), jnp.float32),
        ] + [pltpu.SemaphoreType.DMA] * (2 * NBUF),
    )
    def k(y_hbm, src_hbm, dst_hbm, z_hbm, out_hbm, acc, srcs, dsts, rows, *sems):
        gsem, ssem = sems[:NBUF], sems[NBUF:]
        cid = lax.axis_index("core")
        sid = lax.axis_index("subcore")
        w = cid * NS + sid
        pltpu.sync_copy(src_hbm.at[w], srcs)
        pltpu.sync_copy(dst_hbm.at[w], dsts)
        pltpu.sync_copy(
            z_hbm.at[pl.ds(sid * RPS, RPS)], acc.at[pl.ds(sid * RPS, RPS)]
        )
        plsc.subcore_barrier()

        def g_start(j, b):
            pltpu.async_copy(y_hbm.at[srcs.at[j]], rows.at[b], gsem[b])

        def g_wait(b):
            pltpu.make_async_copy(y_hbm.at[srcs.at[0]], rows.at[b], gsem[b]).wait()

        def s_start(j, b):
            pltpu.async_copy(rows.at[b], acc.at[dsts.at[j]], ssem[b], add=True)

        def s_wait(b):
            pltpu.make_async_copy(rows.at[b], acc.at[dsts.at[0]], ssem[b]).wait()

        for b in range(NBUF):
            g_start(b, b)

        @pl.loop(0, WPS - NBUF, step=NBUF)
        def _(j):
            for b in range(NBUF):
                g_wait(b)
                s_start(j + b, b)
            for b in range(NBUF):
                s_wait(b)
                g_start(j + NBUF + b, b)

        for b in range(NBUF):
            g_wait(b)
            s_start(WPS - NBUF + b, b)
        for b in range(NBUF):
            s_wait(b)

        plsc.subcore_barrier()
        pltpu.sync_copy(
            acc.at[pl.ds(sid * RPS, RPS)], out_hbm.at[cid, pl.ds(sid * RPS, RPS)]
        )

    return k(y, src_w, dst_w, zeros_hbm)


def _pre_kernel(x_ref, w_ref, d0_ref, d1_ref, y_ref, dinv_ref):
    deg = d0_ref[...] + d1_ref[...] + 1.0
    dinv = lax.rsqrt(deg)
    dinv_ref[...] = dinv
    y_ref[...] = jnp.dot(
        x_ref[...], w_ref[...], preferred_element_type=jnp.float32,
        precision=lax.Precision.HIGHEST,
    ) * dinv


def _tc_pre(x, W1, d0, d1):
    return pl.pallas_call(
        _pre_kernel,
        grid=(GRID,),
        in_specs=[
            pl.BlockSpec((BLK, 128), lambda i: (i, 0)),
            pl.BlockSpec((128, 16), lambda i: (0, 0)),
            pl.BlockSpec((BLK, 1), lambda i: (i, 0)),
            pl.BlockSpec((BLK, 1), lambda i: (i, 0)),
        ],
        out_specs=[
            pl.BlockSpec((BLK, 16), lambda i: (i, 0)),
            pl.BlockSpec((BLK, 1), lambda i: (i, 0)),
        ],
        out_shape=[
            jax.ShapeDtypeStruct((N_NODES, 16), jnp.float32),
            jax.ShapeDtypeStruct((N_NODES, 1), jnp.float32),
        ],
    )(x, W1, d0, d1)


def _mid_kernel(s0_ref, s1_ref, y1_ref, dinv_ref, b1_ref, y2_ref):
    dinv = dinv_ref[...]
    h = dinv * (s0_ref[0] + s1_ref[0] + y1_ref[...]) + b1_ref[...]
    h = jnp.maximum(h, 0.0)
    y2_ref[...] = h * dinv


def _tc_mid(s1p, y1, dinv, b1):
    return pl.pallas_call(
        _mid_kernel,
        grid=(GRID,),
        in_specs=[
            pl.BlockSpec((1, BLK, 16), lambda i: (0, i, 0)),
            pl.BlockSpec((1, BLK, 16), lambda i: (1, i, 0)),
            pl.BlockSpec((BLK, 16), lambda i: (i, 0)),
            pl.BlockSpec((BLK, 1), lambda i: (i, 0)),
            pl.BlockSpec((1, 16), lambda i: (0, 0)),
        ],
        out_specs=pl.BlockSpec((BLK, 16), lambda i: (i, 0)),
        out_shape=jax.ShapeDtypeStruct((N_NODES, 16), jnp.float32),
    )(s1p, s1p, y1, dinv, b1)


def _post_kernel(s0_ref, s1_ref, y2_ref, dinv_ref, b2_ref, w2_ref, o_ref):
    agg = dinv_ref[...] * (s0_ref[0] + s1_ref[0] + y2_ref[...])
    z = jnp.dot(
        agg, w2_ref[...], preferred_element_type=jnp.float32,
        precision=lax.Precision.HIGHEST,
    ) + b2_ref[...]
    m = jnp.max(z, axis=1, keepdims=True)
    zs = z - m
    lse = jnp.log(jnp.sum(jnp.exp(zs), axis=1, keepdims=True))
    o_ref[...] = zs - lse


def _tc_post(s2p, y2, dinv, b2, W2):
    return pl.pallas_call(
        _post_kernel,
        grid=(GRID,),
        in_specs=[
            pl.BlockSpec((1, BLK, 16), lambda i: (0, i, 0)),
            pl.BlockSpec((1, BLK, 16), lambda i: (1, i, 0)),
            pl.BlockSpec((BLK, 16), lambda i: (i, 0)),
            pl.BlockSpec((BLK, 1), lambda i: (i, 0)),
            pl.BlockSpec((1, 40), lambda i: (0, 0)),
            pl.BlockSpec((16, 40), lambda i: (0, 0)),
        ],
        out_specs=pl.BlockSpec((BLK, 40), lambda i: (i, 0)),
        out_shape=jax.ShapeDtypeStruct((N_NODES, 40), jnp.float32),
    )(s2p, s2p, y2, dinv, b2, W2)


def kernel(x, edge_index, W1, b1, W2, b2):
    src = edge_index[0].astype(jnp.int32)
    dst = edge_index[1].astype(jnp.int32)
    src_w = jnp.pad(src, (0, EPAD - N_EDGES)).reshape(NW, WPS, WIN)
    dst_w = jnp.pad(dst, (0, EPAD - N_EDGES),
                    constant_values=PAD_ROW).reshape(NW, WPS, WIN)
    z16 = jnp.zeros((NPAD, 16), jnp.float32)

    degp = _sc_degree(dst_w)
    y1, dinv = _tc_pre(x, W1, degp[0, :N_NODES, None], degp[1, :N_NODES, None])
    s1p = _sc_message(y1, src_w, dst_w, z16, 16)
    y2 = _tc_mid(s1p, y1, dinv, b1.reshape(1, 16))
    s2p = _sc_message(y2, src_w, dst_w, z16, 16)
    return _tc_post(s2p, y2, dinv, b2.reshape(1, 40), W2)


# speedup vs baseline: 52.1216x; 1.4060x over previous
import functools

import jax
import jax.numpy as jnp
from jax import lax
from jax.experimental import pallas as pl
from jax.experimental.pallas import tpu as pltpu
from jax.experimental.pallas import tpu_sc as plsc

N_NODES = 10000
NPAD = 10240
PAD_ROW = 10000
N_EDGES = 320000
WIN = 128
NC, NS = 2, 16
NW = NC * NS
WPS = 80
EPAD = NW * WPS * WIN
RPS = NPAD // NS
NBUF = 4
BLK = 1000
GRID = N_NODES // BLK

_mesh = plsc.VectorSubcoreMesh(core_axis_name="core", subcore_axis_name="subcore")
_sc_params = pltpu.CompilerParams(use_tc_tiling_on_sc=False)


def _sc_degree(dst_w):
    @functools.partial(
        pl.kernel,
        out_type=jax.ShapeDtypeStruct((NC, NPAD), jnp.float32),
        mesh=_mesh,
        compiler_params=_sc_params,
        scratch_types=[
            pltpu.VMEM_SHARED((NPAD,), jnp.float32),
            pltpu.VMEM((WPS, WIN), jnp.int32),
            pltpu.VMEM((RPS,), jnp.float32),
            pltpu.VMEM((WIN,), jnp.float32),
            pltpu.SemaphoreType.DMA,
        ],
    )
    def k(dst_hbm, out_hbm, acc, dsts, zbuf, ones_v, ssem):
        cid = lax.axis_index("core")
        sid = lax.axis_index("subcore")
        w = cid * NS + sid
        pltpu.sync_copy(dst_hbm.at[w], dsts)

        @pl.loop(0, RPS // 16)
        def _(i):
            zbuf[pl.ds(i * 16, 16)] = jnp.zeros((16,), jnp.float32)

        pltpu.sync_copy(zbuf, acc.at[pl.ds(sid * RPS, RPS)])

        @pl.loop(0, WIN // 16)
        def _(i):
            ones_v[pl.ds(i * 16, 16)] = jnp.ones((16,), jnp.float32)

        plsc.subcore_barrier()

        @pl.loop(0, WPS)
        def _(j):
            pltpu.async_copy(ones_v, acc.at[dsts.at[j]], ssem, add=True)

        @pl.loop(0, WPS)
        def _(j):
            pltpu.make_async_copy(ones_v, acc.at[dsts.at[0]], ssem).wait()

        plsc.subcore_barrier()
        pltpu.sync_copy(
            acc.at[pl.ds(sid * RPS, RPS)], out_hbm.at[cid, pl.ds(sid * RPS, RPS)]
        )

    return k(dst_w)


def _sc_message(y, src_w, dst_w, zeros_hbm, d):
    @functools.partial(
        pl.kernel,
        out_type=jax.ShapeDtypeStruct((NC, NPAD, d), jnp.float32),
        mesh=_mesh,
        compiler_params=_sc_params,
        scratch_types=[
            pltpu.VMEM_SHARED((NPAD, d), jnp.float32),
            pltpu.VMEM_SHARED((NPAD, d), jnp.float32),
            pltpu.VMEM((WPS, WIN), jnp.int32),
            pltpu.VMEM((WPS, WIN), jnp.int32),
            pltpu.VMEM((NBUF, WIN, d), jnp.float32),
        ] + [pltpu.SemaphoreType.DMA] * (2 * NBUF),
    )
    def k(y_hbm, src_hbm, dst_hbm, z_hbm, out_hbm, acc, ytab, srcs, dsts,
          rows, *sems):
        gsem, ssem = sems[:NBUF], sems[NBUF:]
        cid = lax.axis_index("core")
        sid = lax.axis_index("subcore")
        w = cid * NS + sid
        pltpu.sync_copy(src_hbm.at[w], srcs)
        pltpu.sync_copy(dst_hbm.at[w], dsts)
        pltpu.sync_copy(
            y_hbm.at[pl.ds(sid * RPS, RPS)], ytab.at[pl.ds(sid * RPS, RPS)]
        )
        pltpu.sync_copy(
            z_hbm.at[pl.ds(sid * RPS, RPS)], acc.at[pl.ds(sid * RPS, RPS)]
        )
        plsc.subcore_barrier()

        def g_start(j, b):
            pltpu.async_copy(ytab.at[srcs.at[j]], rows.at[b], gsem[b])

        def g_wait(b):
            pltpu.make_async_copy(ytab.at[srcs.at[0]], rows.at[b], gsem[b]).wait()

        def s_start(j, b):
            pltpu.async_copy(rows.at[b], acc.at[dsts.at[j]], ssem[b], add=True)

        def s_wait(b):
            pltpu.make_async_copy(rows.at[b], acc.at[dsts.at[0]], ssem[b]).wait()

        for b in range(NBUF):
            g_start(b, b)

        @pl.loop(0, WPS - NBUF, step=NBUF)
        def _(j):
            for b in range(NBUF):
                g_wait(b)
                s_start(j + b, b)
            for b in range(NBUF):
                s_wait(b)
                g_start(j + NBUF + b, b)

        for b in range(NBUF):
            g_wait(b)
            s_start(WPS - NBUF + b, b)
        for b in range(NBUF):
            s_wait(b)

        plsc.subcore_barrier()
        pltpu.sync_copy(
            acc.at[pl.ds(sid * RPS, RPS)], out_hbm.at[cid, pl.ds(sid * RPS, RPS)]
        )

    return k(y, src_w, dst_w, zeros_hbm)


def _pre_kernel(x_ref, w_ref, d0_ref, d1_ref, y_ref, dinv_ref):
    deg = d0_ref[...] + d1_ref[...] + 1.0
    dinv = lax.rsqrt(deg)
    dinv_ref[...] = dinv
    y_ref[...] = jnp.dot(
        x_ref[...], w_ref[...], preferred_element_type=jnp.float32,
        precision=lax.Precision.HIGHEST,
    ) * dinv


def _tc_pre(x, W1, d0, d1):
    return pl.pallas_call(
        _pre_kernel,
        grid=(GRID,),
        in_specs=[
            pl.BlockSpec((BLK, 128), lambda i: (i, 0)),
            pl.BlockSpec((128, 16), lambda i: (0, 0)),
            pl.BlockSpec((BLK, 1), lambda i: (i, 0)),
            pl.BlockSpec((BLK, 1), lambda i: (i, 0)),
        ],
        out_specs=[
            pl.BlockSpec((BLK, 16), lambda i: (i, 0)),
            pl.BlockSpec((BLK, 1), lambda i: (i, 0)),
        ],
        out_shape=[
            jax.ShapeDtypeStruct((N_NODES, 16), jnp.float32),
            jax.ShapeDtypeStruct((N_NODES, 1), jnp.float32),
        ],
    )(x, W1, d0, d1)


def _mid_kernel(s0_ref, s1_ref, y1_ref, dinv_ref, b1_ref, y2_ref):
    dinv = dinv_ref[...]
    h = dinv * (s0_ref[0] + s1_ref[0] + y1_ref[...]) + b1_ref[...]
    h = jnp.maximum(h, 0.0)
    y2_ref[...] = h * dinv


def _tc_mid(s1p, y1, dinv, b1):
    return pl.pallas_call(
        _mid_kernel,
        grid=(GRID,),
        in_specs=[
            pl.BlockSpec((1, BLK, 16), lambda i: (0, i, 0)),
            pl.BlockSpec((1, BLK, 16), lambda i: (1, i, 0)),
            pl.BlockSpec((BLK, 16), lambda i: (i, 0)),
            pl.BlockSpec((BLK, 1), lambda i: (i, 0)),
            pl.BlockSpec((1, 16), lambda i: (0, 0)),
        ],
        out_specs=pl.BlockSpec((BLK, 16), lambda i: (i, 0)),
        out_shape=jax.ShapeDtypeStruct((N_NODES, 16), jnp.float32),
    )(s1p, s1p, y1, dinv, b1)


def _post_kernel(s0_ref, s1_ref, y2_ref, dinv_ref, b2_ref, w2_ref, o_ref):
    agg = dinv_ref[...] * (s0_ref[0] + s1_ref[0] + y2_ref[...])
    z = jnp.dot(
        agg, w2_ref[...], preferred_element_type=jnp.float32,
        precision=lax.Precision.HIGHEST,
    ) + b2_ref[...]
    m = jnp.max(z, axis=1, keepdims=True)
    zs = z - m
    lse = jnp.log(jnp.sum(jnp.exp(zs), axis=1, keepdims=True))
    o_ref[...] = zs - lse


def _tc_post(s2p, y2, dinv, b2, W2):
    return pl.pallas_call(
        _post_kernel,
        grid=(GRID,),
        in_specs=[
            pl.BlockSpec((1, BLK, 16), lambda i: (0, i, 0)),
            pl.BlockSpec((1, BLK, 16), lambda i: (1, i, 0)),
            pl.BlockSpec((BLK, 16), lambda i: (i, 0)),
            pl.BlockSpec((BLK, 1), lambda i: (i, 0)),
            pl.BlockSpec((1, 40), lambda i: (0, 0)),
            pl.BlockSpec((16, 40), lambda i: (0, 0)),
        ],
        out_specs=pl.BlockSpec((BLK, 40), lambda i: (i, 0)),
        out_shape=jax.ShapeDtypeStruct((N_NODES, 40), jnp.float32),
    )(s2p, s2p, y2, dinv, b2, W2)


def kernel(x, edge_index, W1, b1, W2, b2):
    src = edge_index[0].astype(jnp.int32)
    dst = edge_index[1].astype(jnp.int32)
    src_w = jnp.pad(src, (0, EPAD - N_EDGES)).reshape(NW, WPS, WIN)
    dst_w = jnp.pad(dst, (0, EPAD - N_EDGES),
                    constant_values=PAD_ROW).reshape(NW, WPS, WIN)
    z16 = jnp.zeros((NPAD, 16), jnp.float32)

    degp = _sc_degree(dst_w)
    y1, dinv = _tc_pre(x, W1, degp[0, :N_NODES, None], degp[1, :N_NODES, None])
    y1p = jnp.pad(y1, ((0, NPAD - N_NODES), (0, 0)))
    s1p = _sc_message(y1p, src_w, dst_w, z16, 16)
    y2 = _tc_mid(s1p, y1, dinv, b1.reshape(1, 16))
    y2p = jnp.pad(y2, ((0, NPAD - N_NODES), (0, 0)))
    s2p = _sc_message(y2p, src_w, dst_w, z16, 16)
    return _tc_post(s2p, y2, dinv, b2.reshape(1, 40), W2)


# speedup vs baseline: 52.6459x; 1.0101x over previous
import functools

import jax
import jax.numpy as jnp
from jax import lax
from jax.experimental import pallas as pl
from jax.experimental.pallas import tpu as pltpu
from jax.experimental.pallas import tpu_sc as plsc

N_NODES = 10000
NPAD = 10240
PAD_ROW = 10000
N_EDGES = 320000
WIN = 128
NC, NS = 2, 16
NW = NC * NS
WPS = 80
EPAD = NW * WPS * WIN
RPS = NPAD // NS
NBUF = 4
BLK = 1000
GRID = N_NODES // BLK

_mesh = plsc.VectorSubcoreMesh(core_axis_name="core", subcore_axis_name="subcore")
_sc_params = pltpu.CompilerParams(use_tc_tiling_on_sc=False)


def _sc_degree(dst_w):
    @functools.partial(
        pl.kernel,
        out_type=jax.ShapeDtypeStruct((NC, NPAD), jnp.float32),
        mesh=_mesh,
        compiler_params=_sc_params,
        scratch_types=[
            pltpu.VMEM_SHARED((NPAD,), jnp.float32),
            pltpu.VMEM((WPS, WIN), jnp.int32),
            pltpu.VMEM((RPS,), jnp.float32),
            pltpu.VMEM((WIN,), jnp.float32),
            pltpu.SemaphoreType.DMA,
        ],
    )
    def k(dst_hbm, out_hbm, acc, dsts, zbuf, ones_v, ssem):
        cid = lax.axis_index("core")
        sid = lax.axis_index("subcore")
        w = cid * NS + sid
        pltpu.sync_copy(dst_hbm.at[w], dsts)

        @pl.loop(0, RPS // 16)
        def _(i):
            zbuf[pl.ds(i * 16, 16)] = jnp.zeros((16,), jnp.float32)

        pltpu.sync_copy(zbuf, acc.at[pl.ds(sid * RPS, RPS)])

        @pl.loop(0, WIN // 16)
        def _(i):
            ones_v[pl.ds(i * 16, 16)] = jnp.ones((16,), jnp.float32)

        plsc.subcore_barrier()

        @pl.loop(0, WPS)
        def _(j):
            pltpu.async_copy(ones_v, acc.at[dsts.at[j]], ssem, add=True)

        @pl.loop(0, WPS)
        def _(j):
            pltpu.make_async_copy(ones_v, acc.at[dsts.at[0]], ssem).wait()

        plsc.subcore_barrier()
        pltpu.sync_copy(
            acc.at[pl.ds(sid * RPS, RPS)], out_hbm.at[cid, pl.ds(sid * RPS, RPS)]
        )

    return k(dst_w)


def _sc_message(y, src_w, dst_w, zeros_hbm, d):
    @functools.partial(
        pl.kernel,
        out_type=jax.ShapeDtypeStruct((NC, NPAD, d), jnp.float32),
        mesh=_mesh,
        compiler_params=_sc_params,
        scratch_types=[
            pltpu.VMEM_SHARED((NPAD, d), jnp.float32),
            pltpu.VMEM_SHARED((NPAD, d), jnp.float32),
            pltpu.VMEM((WPS, WIN), jnp.int32),
            pltpu.VMEM((WPS, WIN), jnp.int32),
            pltpu.VMEM((NBUF, WIN, d), jnp.float32),
        ] + [pltpu.SemaphoreType.DMA] * (2 * NBUF),
    )
    def k(y_hbm, src_hbm, dst_hbm, z_hbm, out_hbm, acc, ytab, srcs, dsts,
          rows, *sems):
        gsem, ssem = sems[:NBUF], sems[NBUF:]
        cid = lax.axis_index("core")
        sid = lax.axis_index("subcore")
        w = cid * NS + sid
        pltpu.sync_copy(src_hbm.at[w], srcs)
        pltpu.sync_copy(dst_hbm.at[w], dsts)
        pltpu.sync_copy(
            y_hbm.at[pl.ds(sid * RPS, RPS)], ytab.at[pl.ds(sid * RPS, RPS)]
        )
        pltpu.sync_copy(
            z_hbm.at[pl.ds(sid * RPS, RPS)], acc.at[pl.ds(sid * RPS, RPS)]
        )
        plsc.subcore_barrier()

        def g_start(j, b):
            pltpu.async_copy(ytab.at[srcs.at[j]], rows.at[b], gsem[b])

        def g_wait(b):
            pltpu.make_async_copy(ytab.at[srcs.at[0]], rows.at[b], gsem[b]).wait()

        def s_start(j, b):
            pltpu.async_copy(rows.at[b], acc.at[dsts.at[j]], ssem[b], add=True)

        def s_wait(b):
            pltpu.make_async_copy(rows.at[b], acc.at[dsts.at[0]], ssem[b]).wait()

        for b in range(NBUF):
            g_start(b, b)

        @pl.loop(0, WPS - NBUF, step=NBUF)
        def _(j):
            for b in range(NBUF):
                g_wait(b)
                s_start(j + b, b)
            for b in range(NBUF):
                s_wait(b)
                g_start(j + NBUF + b, b)

        for b in range(NBUF):
            g_wait(b)
            s_start(WPS - NBUF + b, b)
        for b in range(NBUF):
            s_wait(b)

        plsc.subcore_barrier()
        pltpu.sync_copy(
            acc.at[pl.ds(sid * RPS, RPS)], out_hbm.at[cid, pl.ds(sid * RPS, RPS)]
        )

    return k(y, src_w, dst_w, zeros_hbm)


def _mm_kernel(x_ref, w_ref, u_ref):
    u_ref[...] = jnp.dot(
        x_ref[...], w_ref[...], preferred_element_type=jnp.float32,
        precision=lax.Precision.HIGHEST,
    )


def _tc_matmul(x, W1):
    return pl.pallas_call(
        _mm_kernel,
        grid=(GRID,),
        in_specs=[
            pl.BlockSpec((BLK, 128), lambda i: (i, 0)),
            pl.BlockSpec((128, 16), lambda i: (0, 0)),
        ],
        out_specs=pl.BlockSpec((BLK, 16), lambda i: (i, 0)),
        out_shape=jax.ShapeDtypeStruct((N_NODES, 16), jnp.float32),
    )(x, W1)


def _pre_kernel(u_ref, d0_ref, d1_ref, y_ref, dinv_ref):
    deg = d0_ref[...] + d1_ref[...] + 1.0
    dinv = lax.rsqrt(deg)
    dinv_ref[...] = dinv
    y_ref[...] = u_ref[...] * dinv


def _tc_pre(u, d0, d1):
    return pl.pallas_call(
        _pre_kernel,
        grid=(GRID,),
        in_specs=[
            pl.BlockSpec((BLK, 16), lambda i: (i, 0)),
            pl.BlockSpec((BLK, 1), lambda i: (i, 0)),
            pl.BlockSpec((BLK, 1), lambda i: (i, 0)),
        ],
        out_specs=[
            pl.BlockSpec((BLK, 16), lambda i: (i, 0)),
            pl.BlockSpec((BLK, 1), lambda i: (i, 0)),
        ],
        out_shape=[
            jax.ShapeDtypeStruct((NPAD, 16), jnp.float32),
            jax.ShapeDtypeStruct((N_NODES, 1), jnp.float32),
        ],
    )(u, d0, d1)


def _mid_kernel(s0_ref, s1_ref, y1_ref, dinv_ref, b1_ref, y2_ref):
    dinv = dinv_ref[...]
    h = dinv * (s0_ref[0] + s1_ref[0] + y1_ref[...]) + b1_ref[...]
    h = jnp.maximum(h, 0.0)
    y2_ref[...] = h * dinv


def _tc_mid(s1p, y1, dinv, b1):
    return pl.pallas_call(
        _mid_kernel,
        grid=(GRID,),
        in_specs=[
            pl.BlockSpec((1, BLK, 16), lambda i: (0, i, 0)),
            pl.BlockSpec((1, BLK, 16), lambda i: (1, i, 0)),
            pl.BlockSpec((BLK, 16), lambda i: (i, 0)),
            pl.BlockSpec((BLK, 1), lambda i: (i, 0)),
            pl.BlockSpec((1, 16), lambda i: (0, 0)),
        ],
        out_specs=pl.BlockSpec((BLK, 16), lambda i: (i, 0)),
        out_shape=jax.ShapeDtypeStruct((NPAD, 16), jnp.float32),
    )(s1p, s1p, y1, dinv, b1)


def _post_kernel(s0_ref, s1_ref, y2_ref, dinv_ref, b2_ref, w2_ref, o_ref):
    agg = dinv_ref[...] * (s0_ref[0] + s1_ref[0] + y2_ref[...])
    z = jnp.dot(
        agg, w2_ref[...], preferred_element_type=jnp.float32,
        precision=lax.Precision.HIGHEST,
    ) + b2_ref[...]
    m = jnp.max(z, axis=1, keepdims=True)
    zs = z - m
    lse = jnp.log(jnp.sum(jnp.exp(zs), axis=1, keepdims=True))
    o_ref[...] = zs - lse


def _tc_post(s2p, y2, dinv, b2, W2):
    return pl.pallas_call(
        _post_kernel,
        grid=(GRID,),
        in_specs=[
            pl.BlockSpec((1, BLK, 16), lambda i: (0, i, 0)),
            pl.BlockSpec((1, BLK, 16), lambda i: (1, i, 0)),
            pl.BlockSpec((BLK, 16), lambda i: (i, 0)),
            pl.BlockSpec((BLK, 1), lambda i: (i, 0)),
            pl.BlockSpec((1, 40), lambda i: (0, 0)),
            pl.BlockSpec((16, 40), lambda i: (0, 0)),
        ],
        out_specs=pl.BlockSpec((BLK, 40), lambda i: (i, 0)),
        out_shape=jax.ShapeDtypeStruct((N_NODES, 40), jnp.float32),
    )(s2p, s2p, y2, dinv, b2, W2)


def kernel(x, edge_index, W1, b1, W2, b2):
    src = edge_index[0].astype(jnp.int32)
    dst = edge_index[1].astype(jnp.int32)
    src_w = jnp.pad(src, (0, EPAD - N_EDGES)).reshape(NW, WPS, WIN)
    dst_w = jnp.pad(dst, (0, EPAD - N_EDGES),
                    constant_values=PAD_ROW).reshape(NW, WPS, WIN)
    z16 = jnp.zeros((NPAD, 16), jnp.float32)

    degp = _sc_degree(dst_w)
    u = _tc_matmul(x, W1)
    y1, dinv = _tc_pre(u, degp[0, :N_NODES, None], degp[1, :N_NODES, None])
    s1p = _sc_message(y1, src_w, dst_w, z16, 16)
    y2 = _tc_mid(s1p, y1, dinv, b1.reshape(1, 16))
    s2p = _sc_message(y2, src_w, dst_w, z16, 16)
    return _tc_post(s2p, y2, dinv, b2.reshape(1, 40), W2)


# speedup vs baseline: 53.0476x; 1.0076x over previous
import functools

import jax
import jax.numpy as jnp
from jax import lax
from jax.experimental import pallas as pl
from jax.experimental.pallas import tpu as pltpu
from jax.experimental.pallas import tpu_sc as plsc

N_NODES = 10000
NPAD = 10240
PAD_ROW = 10000
N_EDGES = 320000
WIN = 128
NC, NS = 2, 16
NW = NC * NS
WPS = 80
EPAD = NW * WPS * WIN
RPS = NPAD // NS
NBUF = 4
BLK = 1000
GRID = N_NODES // BLK

_mesh = plsc.VectorSubcoreMesh(core_axis_name="core", subcore_axis_name="subcore")
_sc_params = pltpu.CompilerParams(
    use_tc_tiling_on_sc=False, needs_layout_passes=False
)


def _ring_msg_pass(ytab, acc, srcs, dsts, rows, gsem, ssem):
    def g_start(j, b):
        pltpu.async_copy(ytab.at[srcs.at[j]], rows.at[b], gsem[b])

    def g_wait(b):
        pltpu.make_async_copy(ytab.at[srcs.at[0]], rows.at[b], gsem[b]).wait()

    def s_start(j, b):
        pltpu.async_copy(rows.at[b], acc.at[dsts.at[j]], ssem[b], add=True)

    def s_wait(b):
        pltpu.make_async_copy(rows.at[b], acc.at[dsts.at[0]], ssem[b]).wait()

    for b in range(NBUF):
        g_start(b, b)

    @pl.loop(0, WPS - NBUF, step=NBUF)
    def _(j):
        for b in range(NBUF):
            g_wait(b)
            s_start(j + b, b)
        for b in range(NBUF):
            s_wait(b)
            g_start(j + NBUF + b, b)

    for b in range(NBUF):
        g_wait(b)
        s_start(WPS - NBUF + b, b)
    for b in range(NBUF):
        s_wait(b)


def _sc_layer1(u, src_w, dst_w, zeros_hbm):
    @functools.partial(
        pl.kernel,
        out_type=[
            jax.ShapeDtypeStruct((NC, NPAD, 16), jnp.float32),
            jax.ShapeDtypeStruct((NPAD, 16), jnp.float32),
        ],
        mesh=_mesh,
        compiler_params=_sc_params,
        scratch_types=[
            pltpu.VMEM_SHARED((NPAD,), jnp.float32),
            pltpu.VMEM_SHARED((NPAD, 16), jnp.float32),
            pltpu.VMEM_SHARED((NPAD, 16), jnp.float32),
            pltpu.VMEM((2, WPS, WIN), jnp.int32),
            pltpu.VMEM((WPS, WIN), jnp.int32),
            pltpu.VMEM((WPS, WIN), jnp.int32),
            pltpu.VMEM((RPS,), jnp.float32),
            pltpu.VMEM((RPS,), jnp.float32),
            pltpu.VMEM((RPS, 16), jnp.float32),
            pltpu.VMEM((RPS, 16), jnp.float32),
            pltpu.VMEM((WIN,), jnp.float32),
            pltpu.VMEM((NBUF, WIN, 16), jnp.float32),
            pltpu.SemaphoreType.DMA,
        ] + [pltpu.SemaphoreType.DMA] * (2 * NBUF),
    )
    def k(u_hbm, src_hbm, dst_hbm, z_hbm, out_hbm, dinv_hbm,
          degtab, ytab, acc, degd, srcs, dsts, deg_v, dinv_v, uy_v, drep_v,
          ones_v, rows, dsem, *sems):
        gsem, ssem = sems[:NBUF], sems[NBUF:]
        cid = lax.axis_index("core")
        sid = lax.axis_index("subcore")
        w = cid * NS + sid
        st = pl.ds(sid * RPS, RPS)
        pltpu.sync_copy(dst_hbm.at[pl.ds(sid * 2, 2)], degd)
        pltpu.sync_copy(src_hbm.at[w], srcs)
        pltpu.sync_copy(dst_hbm.at[w], dsts)

        @pl.loop(0, RPS // 16)
        def _(i):
            deg_v[pl.ds(i * 16, 16)] = jnp.zeros((16,), jnp.float32)

        pltpu.sync_copy(deg_v, degtab.at[st])

        @pl.loop(0, WIN // 16)
        def _(i):
            ones_v[pl.ds(i * 16, 16)] = jnp.ones((16,), jnp.float32)

        plsc.subcore_barrier()

        for q in range(2):
            @pl.loop(0, WPS)
            def _(j):
                pltpu.async_copy(ones_v, degtab.at[degd.at[q, j]], dsem, add=True)

        @pl.loop(0, 2 * WPS)
        def _(j):
            pltpu.make_async_copy(ones_v, degtab.at[degd.at[0, 0]], dsem).wait()

        plsc.subcore_barrier()

        pltpu.sync_copy(degtab.at[st], deg_v)

        @pl.loop(0, RPS // 16)
        def _(i):
            c = pl.ds(i * 16, 16)
            xv = deg_v[c] + 1.0
            sv = 0.5 * (xv + 1.0)
            for _it in range(15):
                sv = 0.5 * (sv + xv / sv)
            dinv_v[c] = 1.0 / sv

        pltpu.sync_copy(u_hbm.at[st], uy_v)

        lane = lax.iota(jnp.int32, 16)

        @pl.loop(0, RPS // 16)
        def _(i):
            chunk = dinv_v[pl.ds(i * 16, 16)]
            for j in range(16):
                d_s = jnp.sum(jnp.where(lane == j, chunk, 0.0))
                r = i * 16 + j
                uy_v[r] = uy_v[r] * d_s
                drep_v[r] = jnp.zeros((16,), jnp.float32) + d_s

        pltpu.sync_copy(uy_v, ytab.at[st])

        @pl.when(cid == 0)
        def _():
            pltpu.sync_copy(uy_v, acc.at[st])
            pltpu.sync_copy(drep_v, dinv_hbm.at[st])

        @pl.when(cid == 1)
        def _():
            pltpu.sync_copy(z_hbm.at[st], acc.at[st])

        plsc.subcore_barrier()
        _ring_msg_pass(ytab, acc, srcs, dsts, rows, gsem, ssem)
        plsc.subcore_barrier()
        pltpu.sync_copy(acc.at[st], out_hbm.at[cid, st])

    return k(u, src_w, dst_w, zeros_hbm)


def _sc_layer2(y, src_w, dst_w, zeros_hbm):
    @functools.partial(
        pl.kernel,
        out_type=jax.ShapeDtypeStruct((NC, NPAD, 16), jnp.float32),
        mesh=_mesh,
        compiler_params=_sc_params,
        scratch_types=[
            pltpu.VMEM_SHARED((NPAD, 16), jnp.float32),
            pltpu.VMEM_SHARED((NPAD, 16), jnp.float32),
            pltpu.VMEM((WPS, WIN), jnp.int32),
            pltpu.VMEM((WPS, WIN), jnp.int32),
            pltpu.VMEM((RPS, 16), jnp.float32),
            pltpu.VMEM((NBUF, WIN, 16), jnp.float32),
        ] + [pltpu.SemaphoreType.DMA] * (2 * NBUF),
    )
    def k(y_hbm, src_hbm, dst_hbm, z_hbm, out_hbm, ytab, acc, srcs, dsts,
          yv, rows, *sems):
        gsem, ssem = sems[:NBUF], sems[NBUF:]
        cid = lax.axis_index("core")
        sid = lax.axis_index("subcore")
        w = cid * NS + sid
        st = pl.ds(sid * RPS, RPS)
        pltpu.sync_copy(src_hbm.at[w], srcs)
        pltpu.sync_copy(dst_hbm.at[w], dsts)
        pltpu.sync_copy(y_hbm.at[st], yv)
        pltpu.sync_copy(yv, ytab.at[st])

        @pl.when(cid == 0)
        def _():
            pltpu.sync_copy(yv, acc.at[st])

        @pl.when(cid == 1)
        def _():
            pltpu.sync_copy(z_hbm.at[st], acc.at[st])

        plsc.subcore_barrier()
        _ring_msg_pass(ytab, acc, srcs, dsts, rows, gsem, ssem)
        plsc.subcore_barrier()
        pltpu.sync_copy(acc.at[st], out_hbm.at[cid, st])

    return k(y, src_w, dst_w, zeros_hbm)


def _mm_kernel(x_ref, w_ref, u_ref):
    u_ref[...] = jnp.dot(
        x_ref[...], w_ref[...], preferred_element_type=jnp.float32,
        precision=lax.Precision.HIGHEST,
    )


def _tc_matmul(x, W1):
    return pl.pallas_call(
        _mm_kernel,
        grid=(GRID,),
        in_specs=[
            pl.BlockSpec((BLK, 128), lambda i: (i, 0)),
            pl.BlockSpec((128, 16), lambda i: (0, 0)),
        ],
        out_specs=pl.BlockSpec((BLK, 16), lambda i: (i, 0)),
        out_shape=jax.ShapeDtypeStruct((NPAD, 16), jnp.float32),
    )(x, W1)


def _mid_kernel(s0_ref, s1_ref, dinv_ref, b1_ref, y2_ref):
    dinv = dinv_ref[...]
    h = dinv * (s0_ref[0] + s1_ref[0]) + b1_ref[...]
    h = jnp.maximum(h, 0.0)
    y2_ref[...] = h * dinv


def _tc_mid(s1p, dinv16, b1):
    return pl.pallas_call(
        _mid_kernel,
        grid=(GRID,),
        in_specs=[
            pl.BlockSpec((1, BLK, 16), lambda i: (0, i, 0)),
            pl.BlockSpec((1, BLK, 16), lambda i: (1, i, 0)),
            pl.BlockSpec((BLK, 16), lambda i: (i, 0)),
            pl.BlockSpec((1, 16), lambda i: (0, 0)),
        ],
        out_specs=pl.BlockSpec((BLK, 16), lambda i: (i, 0)),
        out_shape=jax.ShapeDtypeStruct((NPAD, 16), jnp.float32),
    )(s1p, s1p, dinv16, b1)


def _post_kernel(s0_ref, s1_ref, dinv_ref, b2_ref, w2_ref, o_ref):
    agg = dinv_ref[...] * (s0_ref[0] + s1_ref[0])
    z = jnp.dot(
        agg, w2_ref[...], preferred_element_type=jnp.float32,
        precision=lax.Precision.HIGHEST,
    ) + b2_ref[...]
    m = jnp.max(z, axis=1, keepdims=True)
    zs = z - m
    lse = jnp.log(jnp.sum(jnp.exp(zs), axis=1, keepdims=True))
    o_ref[...] = zs - lse


def _tc_post(s2p, dinv16, b2, W2):
    return pl.pallas_call(
        _post_kernel,
        grid=(GRID,),
        in_specs=[
            pl.BlockSpec((1, BLK, 16), lambda i: (0, i, 0)),
            pl.BlockSpec((1, BLK, 16), lambda i: (1, i, 0)),
            pl.BlockSpec((BLK, 16), lambda i: (i, 0)),
            pl.BlockSpec((1, 40), lambda i: (0, 0)),
            pl.BlockSpec((16, 40), lambda i: (0, 0)),
        ],
        out_specs=pl.BlockSpec((BLK, 40), lambda i: (i, 0)),
        out_shape=jax.ShapeDtypeStruct((N_NODES, 40), jnp.float32),
    )(s2p, s2p, dinv16, b2, W2)


def kernel(x, edge_index, W1, b1, W2, b2):
    src = edge_index[0].astype(jnp.int32)
    dst = edge_index[1].astype(jnp.int32)
    src_w = jnp.pad(src, (0, EPAD - N_EDGES)).reshape(NW, WPS, WIN)
    dst_p = jnp.pad(dst, (0, EPAD - N_EDGES), constant_values=PAD_ROW)
    dst_w = dst_p.reshape(NW, WPS, WIN)
    z16 = jnp.zeros((NPAD, 16), jnp.float32)

    u = _tc_matmul(x, W1)
    s1p, dinv16 = _sc_layer1(u, src_w, dst_w, z16)
    y2 = _tc_mid(s1p, dinv16, b1.reshape(1, 16))
    s2p = _sc_layer2(y2, src_w, dst_w, z16)
    return _tc_post(s2p, dinv16, b2.reshape(1, 40), W2)
